# baseline (speedup 1.0000x reference)
import functools

import jax
import jax.numpy as jnp
from jax import lax
from jax.experimental import pallas as pl
from jax.experimental.pallas import tpu as pltpu
from jax.experimental.pallas import tpu_sc as plsc

NUM_GRAPHS = 1024
N_ROWS = 100000
D = 128
NC = 2
NS = 16
NW = NC * NS
LANES = 16

SLICE = 3200
LAST = N_ROWS - (NW - 1) * SLICE

SLABS = 39
BLK = SLABS * D
GRID = -(-N_ROWS // BLK)
N_PAD = GRID * BLK


def _mask_body(b_hbm, drop_hbm, mask_hbm, drop_v, idx_v, mask_v):
    wid = lax.axis_index("s") * NC + lax.axis_index("c")
    pltpu.sync_copy(drop_hbm, drop_v)
    base = wid * SLICE

    def gather_slice(rows):
        pltpu.sync_copy(b_hbm.at[pl.ds(base, rows)], idx_v.at[pl.ds(0, rows)])

        @plsc.parallel_loop(0, rows // LANES)
        def gather_group(g):
            sl = pl.ds(g * LANES, LANES)
            mask_v[sl] = plsc.load_gather(drop_v, [idx_v[sl]])

        pltpu.sync_copy(mask_v.at[pl.ds(0, rows)],
                        mask_hbm.at[pl.ds(base, rows)])

    @pl.when(wid < NW - 1)
    def _full():
        gather_slice(SLICE)

    @pl.when(wid == NW - 1)
    def _last():
        gather_slice(LAST)


def _scale_body(m_ref, x_ref, o_ref):
    i = pl.program_id(0)
    mt = jnp.transpose(m_ref[i])
    for s in range(SLABS):
        col = mt[:, s:s + 1]
        sl = pl.ds(s * D, D)
        o_ref[sl, :] = x_ref[sl, :] * col


def kernel(x, batch):
    drop = jnp.ones((NUM_GRAPHS,), x.dtype)
    batch32 = batch.astype(jnp.int32)
    mesh = plsc.VectorSubcoreMesh(core_axis_name="c", subcore_axis_name="s")
    mask = functools.partial(
        pl.kernel,
        mesh=mesh,
        out_type=jax.ShapeDtypeStruct((N_PAD,), jnp.float32),
        compiler_params=pltpu.CompilerParams(needs_layout_passes=False),
        scratch_types=[
            pltpu.VMEM((NUM_GRAPHS,), jnp.float32),
            pltpu.VMEM((SLICE,), jnp.int32),
            pltpu.VMEM((SLICE,), jnp.float32),
        ],
    )(_mask_body)(batch32, drop)

    return pl.pallas_call(
        _scale_body,
        grid=(GRID,),
        in_specs=[
            pl.BlockSpec((GRID, SLABS, D), lambda i: (0, 0, 0)),
            pl.BlockSpec((BLK, D), lambda i: (i, 0)),
        ],
        out_specs=pl.BlockSpec((BLK, D), lambda i: (i, 0)),
        out_shape=jax.ShapeDtypeStruct((N_ROWS, D), x.dtype),
        compiler_params=pltpu.CompilerParams(
            dimension_semantics=("arbitrary",),
        ),
    )(mask.reshape(GRID, SLABS, D), x)


# speedup vs baseline: 1.0999x; 1.0999x over previous
import functools

import jax
import jax.numpy as jnp
from jax import lax
from jax.experimental import pallas as pl
from jax.experimental.pallas import tpu as pltpu
from jax.experimental.pallas import tpu_sc as plsc

NUM_GRAPHS = 1024
N_ROWS = 100000
D = 128
CHUNK = 400
NUM_CHUNKS = N_ROWS // CHUNK
NC = 2
NS = 16
NW = NC * NS
LANES = 16
GROUPS = CHUNK // LANES


def _body(x_hbm, b_hbm, drop_hbm, out_hbm, drop_v, idx_v, buf_v,
          ix_sem, ib_sem, out_sem):
    wid = lax.axis_index("s") * NC + lax.axis_index("c")
    pltpu.sync_copy(drop_hbm, drop_v)

    def base_of(k):
        return (k * NW + wid) * CHUNK

    def start_in(k, b):
        pltpu.async_copy(x_hbm.at[pl.ds(base_of(k), CHUNK)],
                         buf_v.at[pl.ds(b * CHUNK, CHUNK)], ix_sem.at[b])
        pltpu.async_copy(b_hbm.at[pl.ds(base_of(k), CHUNK)],
                         idx_v.at[pl.ds(b * CHUNK, CHUNK)], ib_sem.at[b])

    def wait_in(b):
        pltpu.make_async_copy(x_hbm.at[pl.ds(0, CHUNK)],
                              buf_v.at[pl.ds(b * CHUNK, CHUNK)],
                              ix_sem.at[b]).wait()
        pltpu.make_async_copy(b_hbm.at[pl.ds(0, CHUNK)],
                              idx_v.at[pl.ds(b * CHUNK, CHUNK)],
                              ib_sem.at[b]).wait()

    def start_out(k, b):
        pltpu.async_copy(buf_v.at[pl.ds(b * CHUNK, CHUNK)],
                         out_hbm.at[pl.ds(base_of(k), CHUNK)], out_sem.at[b])

    def wait_out(b):
        pltpu.make_async_copy(buf_v.at[pl.ds(b * CHUNK, CHUNK)],
                              out_hbm.at[pl.ds(0, CHUNK)],
                              out_sem.at[b]).wait()

    def compute(b):
        @plsc.parallel_loop(0, GROUPS)
        def scale_group(g):
            iv = idx_v[pl.ds(b * CHUNK + g * LANES, LANES)]
            mvec = plsc.load_gather(drop_v, [iv])
            for r in range(LANES):
                m = mvec[r]
                for j in range(D // LANES):
                    sl = pl.ds(j * LANES, LANES)
                    row = b * CHUNK + g * LANES + r
                    buf_v[row, sl] = buf_v[row, sl] * m

    n_mine = 7 + jnp.where(wid < NUM_CHUNKS - 7 * NW, 1, 0)

    start_in(0, 0)

    def chunk_step(k, _):
        b = k & 1
        wait_in(b)

        @pl.when(jnp.logical_and(k + 1 < n_mine, k >= 1))
        def _wait_prev_out():
            wait_out(1 - b)

        @pl.when(k + 1 < n_mine)
        def _prefetch_next():
            start_in(k + 1, 1 - b)

        compute(b)
        start_out(k, b)
        return 0

    lax.fori_loop(0, n_mine, chunk_step, 0)
    wait_out(0)
    wait_out(1)


def kernel(x, batch):
    drop = jnp.ones((NUM_GRAPHS,), x.dtype)
    batch32 = batch.astype(jnp.int32)
    mesh = plsc.VectorSubcoreMesh(core_axis_name="c", subcore_axis_name="s")
    run = functools.partial(
        pl.kernel,
        mesh=mesh,
        out_type=jax.ShapeDtypeStruct((N_ROWS, D), x.dtype),
        compiler_params=pltpu.CompilerParams(needs_layout_passes=False),
        scratch_types=[
            pltpu.VMEM((NUM_GRAPHS,), jnp.float32),
            pltpu.VMEM((2 * CHUNK,), jnp.int32),
            pltpu.VMEM((2 * CHUNK, D), jnp.float32),
            pltpu.SemaphoreType.DMA((2,)),
            pltpu.SemaphoreType.DMA((2,)),
            pltpu.SemaphoreType.DMA((2,)),
        ],
    )(_body)
    return run(x, batch32, drop)
